# baseline (speedup 1.0000x reference)
import functools

import jax
import jax.numpy as jnp
from jax import lax
from jax.experimental import pallas as pl
from jax.experimental.pallas import tpu as pltpu
from jax.experimental.pallas import tpu_sc as plsc

_W = 128


def _make_gather(V, D, B):
    info = plsc.get_sparse_core_info()
    NC, NS, L = info.num_cores, info.num_subcores, info.num_lanes
    NW = NC * NS
    assert B % (8 * NW) == 0 and D % L == 0 and V % 2 == 0
    b_per_w = B // NW
    nwin = b_per_w // _W
    assert nwin % 2 == 0
    mesh = plsc.VectorSubcoreMesh(core_axis_name="c", subcore_axis_name="s")

    @functools.partial(
        pl.kernel,
        mesh=mesh,
        out_type=jax.ShapeDtypeStruct((B, D), jnp.float32),
        scratch_types=[
            pltpu.VMEM((b_per_w,), jnp.int32),
            pltpu.VMEM((b_per_w,), jnp.int32),
            pltpu.VMEM((_W, 2 * D), jnp.float32),
            pltpu.VMEM((_W, 2 * D), jnp.float32),
            pltpu.VMEM((_W, D), jnp.float32),
            pltpu.SemaphoreType.DMA,
        ],
    )
    def gather_kernel(idx_hbm, pairs_hbm, out_hbm, idx_v, pid_v, buf0, buf1, ow, gsem):
        wid = lax.axis_index("s") * NC + lax.axis_index("c")
        base = wid * b_per_w
        pltpu.sync_copy(idx_hbm.at[pl.ds(base, b_per_w)], idx_v)

        def decompose(i, carry):
            vec = idx_v[pl.ds(i * L, L)]
            pid_v[pl.ds(i * L, L)] = lax.shift_right_logical(vec, 1)
            return carry

        lax.fori_loop(0, b_per_w // L, decompose, 0)

        def start_gather(w, buf):
            pltpu.async_copy(pairs_hbm.at[pid_v.at[pl.ds(w * _W, _W)]], buf, gsem)

        def wait_gather(buf):
            pltpu.make_async_copy(pairs_hbm.at[pl.ds(0, _W)], buf, gsem).wait()

        def extract_and_emit(w, buf):
            for g in range(_W // L):
                vec = idx_v[pl.ds(w * _W + g * L, L)]
                for k in range(L):
                    h = (vec[k] & 1) * D
                    r = g * L + k
                    for c in range(D // L):
                        ow[r, pl.ds(c * L, L)] = buf[r, pl.ds(h + c * L, L)]
            pltpu.sync_copy(ow, out_hbm.at[pl.ds(base + w * _W, _W)])

        start_gather(0, buf0)

        def pair(p, carry):
            w0 = 2 * p
            wait_gather(buf0)
            start_gather(w0 + 1, buf1)
            extract_and_emit(w0, buf0)
            wait_gather(buf1)

            @pl.when(p < nwin // 2 - 1)
            def _():
                start_gather(w0 + 2, buf0)

            extract_and_emit(w0 + 1, buf1)
            return carry

        lax.fori_loop(0, nwin // 2, pair, 0)

    return gather_kernel


def kernel(input, embed_table):
    B = input.shape[0]
    V, D = embed_table.shape
    H = V // 2
    idx = input.astype(jnp.int32)
    pairs = jnp.concatenate(
        [
            embed_table[:H].reshape(H // 2, 2 * D),
            embed_table[H:].reshape(H // 2, 2 * D),
        ],
        axis=0,
    )
    return _make_gather(V, D, B)(idx, pairs)


# speedup vs baseline: 2.9498x; 2.9498x over previous
import functools

import jax
import jax.numpy as jnp
from jax import lax
from jax.experimental import pallas as pl
from jax.experimental.pallas import tpu as pltpu
from jax.experimental.pallas import tpu_sc as plsc


def _make_gather(V, D, B):
    info = plsc.get_sparse_core_info()
    NC, NS = info.num_cores, info.num_subcores
    NW = NC * NS
    assert B % (8 * NW) == 0 and D % info.num_lanes == 0
    b_per_w = B // NW
    mesh = plsc.VectorSubcoreMesh(core_axis_name="c", subcore_axis_name="s")

    @functools.partial(
        pl.kernel,
        mesh=mesh,
        out_type=jax.ShapeDtypeStruct((B, D), jnp.float32),
        scratch_types=[
            pltpu.VMEM((b_per_w,), jnp.int32),
            pltpu.VMEM((b_per_w, D), jnp.float32),
            pltpu.SemaphoreType.DMA,
        ],
    )
    def gather_kernel(idx_hbm, table_hbm, out_hbm, idx_v, rows_v, sem):
        wid = lax.axis_index("s") * NC + lax.axis_index("c")
        base = wid * b_per_w
        pltpu.sync_copy(idx_hbm.at[pl.ds(base, b_per_w)], idx_v)

        def issue(j, carry):
            vec = idx_v[pl.ds(j * 16, 16)]
            for k in range(16):
                pltpu.async_copy(
                    table_hbm.at[pl.ds(vec[k], 1)],
                    rows_v.at[pl.ds(j * 16 + k, 1)],
                    sem,
                )
            return carry

        lax.fori_loop(0, b_per_w // 16, issue, 0)
        pltpu.make_async_copy(
            table_hbm.at[pl.ds(0, b_per_w)], rows_v, sem
        ).wait()
        pltpu.sync_copy(rows_v, out_hbm.at[pl.ds(base, b_per_w)])

    return gather_kernel


def kernel(input, embed_table):
    B = input.shape[0]
    V, D = embed_table.shape
    idx = input.astype(jnp.int32)
    return _make_gather(V, D, B)(idx, embed_table)
